# baseline (speedup 1.0000x reference)
import functools

import jax
import jax.numpy as jnp
from jax import lax
from jax.experimental import pallas as pl
from jax.experimental.pallas import tpu as pltpu
from jax.experimental.pallas import tpu_sc as plsc

NC = 2
NS = 16
K = 125


def _make_sc_agg(N, H, NCHUNK, with_deg):
    mesh = plsc.VectorSubcoreMesh(core_axis_name="c", subcore_axis_name="s",
                                  num_cores=NC, num_subcores=NS)
    RS = -(-N // NS)
    RS += (-RS) % 8

    out_type = [jax.ShapeDtypeStruct((NC, N, H), jnp.float32)]
    scratch = [
        pltpu.VMEM_SHARED((N, H), jnp.float32),
        pltpu.VMEM((4, K), jnp.int32),
        pltpu.VMEM((4, K), jnp.int32),
        pltpu.VMEM((K, H), jnp.float32),
        pltpu.VMEM((K, H), jnp.float32),
        pltpu.SemaphoreType.DMA,
        pltpu.SemaphoreType.DMA,
        pltpu.SemaphoreType.DMA,
        pltpu.SemaphoreType.DMA,
        pltpu.SemaphoreType.DMA,
        pltpu.SemaphoreType.DMA,
    ]
    if with_deg:
        out_type.append(jax.ShapeDtypeStruct((NC, N, 16), jnp.float32))
        scratch += [
            pltpu.VMEM_SHARED((N, 16), jnp.float32),
            pltpu.VMEM((K, 16), jnp.float32),
            pltpu.VMEM((120, 16), jnp.float32),
        ]

    def body(h_hbm, srcr_hbm, dstr_hbm, ones_hbm, *rest):
        if with_deg:
            (part_hbm, degp_hbm,
             agg_s, srcb, dstb, rows0, rows1,
             sg0, sg1, si0, si1, si2, si3,
             deg_s, ones_v, zbuf) = rest
        else:
            (part_hbm,
             agg_s, srcb, dstb, rows0, rows1,
             sg0, sg1, si0, si1, si2, si3) = rest
        rows = (rows0, rows1)
        sg = (sg0, sg1)
        si = (si0, si1, si2, si3)
        c = lax.axis_index("c")
        s = lax.axis_index("s")
        wid = s * NC + c
        row0 = pl.multiple_of(jnp.minimum(s * RS, N - RS), 8)

        def fetch_idx(j, v, sem):
            pltpu.async_copy(srcr_hbm.at[wid, j], srcb.at[v], sem)
            pltpu.async_copy(dstr_hbm.at[wid, j], dstb.at[v], sem)

        def wait_idx(j, v, sem):
            pltpu.make_async_copy(srcr_hbm.at[wid, j], srcb.at[v], sem).wait()
            pltpu.make_async_copy(dstr_hbm.at[wid, j], dstb.at[v], sem).wait()

        for v in range(4):
            fetch_idx(v, v, si[v])
        zv = jnp.zeros((16,), jnp.float32)

        def zrow(i, carry):
            for jj in range(H // 16):
                rows0[i, pl.ds(16 * jj, 16)] = zv
            return carry

        lax.fori_loop(0, 120, zrow, 0)
        for t in range(RS // 120):
            pltpu.sync_copy(rows0.at[pl.ds(0, 120)],
                            agg_s.at[pl.ds(row0 + 120 * t, 120)])
        rem = RS - (RS // 120) * 120
        if rem:
            pltpu.sync_copy(rows0.at[pl.ds(0, rem)],
                            agg_s.at[pl.ds(row0 + (RS // 120) * 120, rem)])
        if with_deg:
            pltpu.sync_copy(ones_hbm, ones_v)

            def zrowd(i, carry):
                zbuf[i, pl.ds(0, 16)] = zv
                return carry

            lax.fori_loop(0, 120, zrowd, 0)
            for t in range(RS // 120):
                pltpu.sync_copy(zbuf.at[pl.ds(0, 120)],
                                deg_s.at[pl.ds(row0 + 120 * t, 120)])
            if rem:
                pltpu.sync_copy(zbuf.at[pl.ds(0, rem)],
                                deg_s.at[pl.ds(row0 + (RS // 120) * 120, rem)])
        plsc.subcore_barrier()
        for v in range(2):
            wait_idx(v, v, si[v])
            pltpu.async_copy(h_hbm.at[srcb.at[v]], rows[v], sg[v])

        def step(g, carry):
            j0 = 4 * g
            for u in range(4):
                j = j0 + u
                b = u % 2
                pltpu.make_async_copy(h_hbm.at[srcb.at[u]], rows[b], sg[b]).wait()
                pltpu.sync_copy(rows[b], agg_s.at[dstb.at[u]], add=True)
                if with_deg:
                    pltpu.sync_copy(ones_v, deg_s.at[dstb.at[u]], add=True)

                @pl.when(j + 4 < NCHUNK)
                def _():
                    fetch_idx(j + 4, u, si[u])

                @pl.when(j + 2 < NCHUNK)
                def _():
                    v2 = (u + 2) % 4
                    wait_idx(j + 2, v2, si[v2])
                    pltpu.async_copy(h_hbm.at[srcb.at[v2]], rows[b], sg[b])
            return carry

        lax.fori_loop(0, NCHUNK // 4, step, 0)
        plsc.subcore_barrier()

        pltpu.sync_copy(agg_s.at[pl.ds(row0, RS)], part_hbm.at[c, pl.ds(row0, RS)])
        if with_deg:
            pltpu.sync_copy(deg_s.at[pl.ds(row0, RS)], degp_hbm.at[c, pl.ds(row0, RS)])

    return pl.kernel(
        body, out_type=out_type, mesh=mesh, scratch_types=scratch,
        compiler_params=pltpu.CompilerParams(use_tc_tiling_on_sc=False))


def _proj_relu_body(x_ref, w_ref, b_ref, o_ref):
    o_ref[...] = jnp.maximum(
        jnp.dot(x_ref[...], w_ref[...], preferred_element_type=jnp.float32)
        + b_ref[...], 0.0)


def _rterm_body(h_ref, w_ref, b_ref, o_ref):
    o_ref[...] = (jnp.dot(h_ref[...], w_ref[...],
                          preferred_element_type=jnp.float32) + b_ref[...])


def _sage_bn_body(part_ref, degp_ref, r_ref, wl_ref, g_ref,
                  be_ref, o_ref):
    agg = part_ref[0] + part_ref[1]
    deg = degp_ref[0, :, 0:1] + degp_ref[1, :, 0:1]
    mean = agg * (1.0 / jnp.maximum(deg, 1.0))
    t = (jnp.dot(mean, wl_ref[...], preferred_element_type=jnp.float32)
         + r_ref[...])
    mu = jnp.mean(t, axis=0, keepdims=True)
    var = jnp.mean((t - mu) * (t - mu), axis=0, keepdims=True)
    o_ref[...] = jnp.maximum(
        (t - mu) * lax.rsqrt(var + 1e-5) * g_ref[...] + be_ref[...], 0.0)


def _sage_bn_proj_body(part_ref, degp_ref, r_ref, wl_ref,
                       g_ref, be_ref, wo_ref, bo_ref, o_ref):
    agg = part_ref[0] + part_ref[1]
    deg = degp_ref[0, :, 0:1] + degp_ref[1, :, 0:1]
    mean = agg * (1.0 / jnp.maximum(deg, 1.0))
    t = (jnp.dot(mean, wl_ref[...], preferred_element_type=jnp.float32)
         + r_ref[...])
    mu = jnp.mean(t, axis=0, keepdims=True)
    var = jnp.mean((t - mu) * (t - mu), axis=0, keepdims=True)
    r = jnp.maximum(
        (t - mu) * lax.rsqrt(var + 1e-5) * g_ref[...] + be_ref[...], 0.0)
    o_ref[...] = (jnp.dot(r, wo_ref[...], preferred_element_type=jnp.float32)
                  + bo_ref[...])


@jax.jit
def kernel(x, edge_index, Wi, bi, Wl0, bl0, Wr0, g0, be0, Wl1, bl1, Wr1, g1,
           be1, Wo, bo):
    N, D = x.shape
    H = Wi.shape[1]
    O = Wo.shape[1]
    E = edge_index.shape[1]
    NW = NC * NS
    assert E % (NW * K) == 0 and (E // (NW * K)) % 4 == 0
    NCHUNK = E // (NW * K)

    srcr = edge_index[0].reshape(NW, NCHUNK, K)
    dstr = edge_index[1].reshape(NW, NCHUNK, K)
    ones = jnp.ones((K, 16), jnp.float32)

    agg_deg = _make_sc_agg(N, H, NCHUNK, with_deg=True)
    agg_only = _make_sc_agg(N, H, NCHUNK, with_deg=False)

    proj = pl.pallas_call(
        _proj_relu_body,
        out_shape=jax.ShapeDtypeStruct((N, H), jnp.float32))
    rterm = pl.pallas_call(
        _rterm_body,
        out_shape=jax.ShapeDtypeStruct((N, H), jnp.float32))
    sage_bn = pl.pallas_call(
        _sage_bn_body,
        out_shape=jax.ShapeDtypeStruct((N, H), jnp.float32))
    sage_bn_proj = pl.pallas_call(
        _sage_bn_proj_body,
        out_shape=jax.ShapeDtypeStruct((N, O), jnp.float32))

    h0 = proj(x, Wi, bi.reshape(1, H))
    part0, degp = agg_deg(h0, srcr, dstr, ones)
    r0 = rterm(h0, Wr0, bl0.reshape(1, H))
    h1 = sage_bn(part0, degp, r0, Wl0,
                 g0.reshape(1, H), be0.reshape(1, H))
    (part1,) = agg_only(h1, srcr, dstr, ones)
    r1 = rterm(h1, Wr1, bl1.reshape(1, H))
    return sage_bn_proj(part1, degp, r1, Wl1,
                        g1.reshape(1, H), be1.reshape(1, H), Wo,
                        bo.reshape(1, O))


# speedup vs baseline: 1.0273x; 1.0273x over previous
import functools

import jax
import jax.numpy as jnp
from jax import lax
from jax.experimental import pallas as pl
from jax.experimental.pallas import tpu as pltpu
from jax.experimental.pallas import tpu_sc as plsc

NC = 2
NS = 16
K = 125


def _make_sc_agg(N, H, NCHUNK, with_deg):
    mesh = plsc.VectorSubcoreMesh(core_axis_name="c", subcore_axis_name="s",
                                  num_cores=NC, num_subcores=NS)
    RS = -(-N // NS)
    RS += (-RS) % 8

    out_type = [jax.ShapeDtypeStruct((NC, N, H), jnp.float32)]
    scratch = [
        pltpu.VMEM_SHARED((N, H), jnp.float32),
        pltpu.VMEM((4, 2, K), jnp.int32),
        pltpu.VMEM((K, H), jnp.float32),
        pltpu.VMEM((K, H), jnp.float32),
        pltpu.SemaphoreType.DMA,
        pltpu.SemaphoreType.DMA,
        pltpu.SemaphoreType.DMA,
        pltpu.SemaphoreType.DMA,
        pltpu.SemaphoreType.DMA,
        pltpu.SemaphoreType.DMA,
    ]
    if with_deg:
        out_type.append(jax.ShapeDtypeStruct((NC, N, 16), jnp.float32))
        scratch += [
            pltpu.VMEM_SHARED((N, 16), jnp.float32),
            pltpu.VMEM((K, 16), jnp.float32),
            pltpu.VMEM((120, 16), jnp.float32),
        ]

    def body(h_hbm, er_hbm, ones_hbm, *rest):
        if with_deg:
            (part_hbm, degp_hbm,
             agg_s, idxb, rows0, rows1,
             sg0, sg1, si0, si1, si2, si3,
             deg_s, ones_v, zbuf) = rest
        else:
            (part_hbm,
             agg_s, idxb, rows0, rows1,
             sg0, sg1, si0, si1, si2, si3) = rest
        rows = (rows0, rows1)
        sg = (sg0, sg1)
        si = (si0, si1, si2, si3)
        c = lax.axis_index("c")
        s = lax.axis_index("s")
        wid = s * NC + c
        row0 = pl.multiple_of(jnp.minimum(s * RS, N - RS), 8)

        def fetch_idx(j, v, sem):
            pltpu.async_copy(er_hbm.at[wid, j], idxb.at[v], sem)

        def wait_idx(j, v, sem):
            pltpu.make_async_copy(er_hbm.at[wid, j], idxb.at[v], sem).wait()

        for v in range(4):
            fetch_idx(v, v, si[v])
        zv = jnp.zeros((16,), jnp.float32)

        def zrow(i, carry):
            for jj in range(H // 16):
                rows0[i, pl.ds(16 * jj, 16)] = zv
            return carry

        lax.fori_loop(0, 120, zrow, 0)
        for t in range(RS // 120):
            pltpu.sync_copy(rows0.at[pl.ds(0, 120)],
                            agg_s.at[pl.ds(row0 + 120 * t, 120)])
        rem = RS - (RS // 120) * 120
        if rem:
            pltpu.sync_copy(rows0.at[pl.ds(0, rem)],
                            agg_s.at[pl.ds(row0 + (RS // 120) * 120, rem)])
        if with_deg:
            pltpu.sync_copy(ones_hbm, ones_v)

            def zrowd(i, carry):
                zbuf[i, pl.ds(0, 16)] = zv
                return carry

            lax.fori_loop(0, 120, zrowd, 0)
            for t in range(RS // 120):
                pltpu.sync_copy(zbuf.at[pl.ds(0, 120)],
                                deg_s.at[pl.ds(row0 + 120 * t, 120)])
            if rem:
                pltpu.sync_copy(zbuf.at[pl.ds(0, rem)],
                                deg_s.at[pl.ds(row0 + (RS // 120) * 120, rem)])
        plsc.subcore_barrier()
        for v in range(2):
            wait_idx(v, v, si[v])
            pltpu.async_copy(h_hbm.at[idxb.at[v, 0]], rows[v], sg[v])

        def step(g, carry):
            j0 = 4 * g
            for u in range(4):
                j = j0 + u
                b = u % 2
                pltpu.make_async_copy(h_hbm.at[idxb.at[u, 0]], rows[b], sg[b]).wait()
                pltpu.sync_copy(rows[b], agg_s.at[idxb.at[u, 1]], add=True)
                if with_deg:
                    pltpu.sync_copy(ones_v, deg_s.at[idxb.at[u, 1]], add=True)

                @pl.when(j + 4 < NCHUNK)
                def _():
                    fetch_idx(j + 4, u, si[u])

                @pl.when(j + 2 < NCHUNK)
                def _():
                    v2 = (u + 2) % 4
                    wait_idx(j + 2, v2, si[v2])
                    pltpu.async_copy(h_hbm.at[idxb.at[v2, 0]], rows[b], sg[b])
            return carry

        lax.fori_loop(0, NCHUNK // 4, step, 0)
        plsc.subcore_barrier()

        pltpu.sync_copy(agg_s.at[pl.ds(row0, RS)], part_hbm.at[c, pl.ds(row0, RS)])
        if with_deg:
            pltpu.sync_copy(deg_s.at[pl.ds(row0, RS)], degp_hbm.at[c, pl.ds(row0, RS)])

    return pl.kernel(
        body, out_type=out_type, mesh=mesh, scratch_types=scratch,
        compiler_params=pltpu.CompilerParams(use_tc_tiling_on_sc=False))


def _proj_relu_body(x_ref, w_ref, b_ref, o_ref):
    o_ref[...] = jnp.maximum(
        jnp.dot(x_ref[...], w_ref[...], preferred_element_type=jnp.float32)
        + b_ref[...], 0.0)


def _sage_bn_body(part_ref, degp_ref, h_ref, wl_ref, wr_ref, bl_ref, g_ref,
                  be_ref, o_ref):
    agg = part_ref[0] + part_ref[1]
    deg = degp_ref[0, :, 0:1] + degp_ref[1, :, 0:1]
    mean = agg * (1.0 / jnp.maximum(deg, 1.0))
    t = (jnp.dot(mean, wl_ref[...], preferred_element_type=jnp.float32)
         + jnp.dot(h_ref[...], wr_ref[...], preferred_element_type=jnp.float32)
         + bl_ref[...])
    mu = jnp.mean(t, axis=0, keepdims=True)
    var = jnp.mean((t - mu) * (t - mu), axis=0, keepdims=True)
    o_ref[...] = jnp.maximum(
        (t - mu) * lax.rsqrt(var + 1e-5) * g_ref[...] + be_ref[...], 0.0)


def _sage_bn_proj_body(part_ref, degp_ref, h_ref, wl_ref, wr_ref, bl_ref,
                       g_ref, be_ref, wo_ref, bo_ref, o_ref):
    agg = part_ref[0] + part_ref[1]
    deg = degp_ref[0, :, 0:1] + degp_ref[1, :, 0:1]
    mean = agg * (1.0 / jnp.maximum(deg, 1.0))
    t = (jnp.dot(mean, wl_ref[...], preferred_element_type=jnp.float32)
         + jnp.dot(h_ref[...], wr_ref[...], preferred_element_type=jnp.float32)
         + bl_ref[...])
    mu = jnp.mean(t, axis=0, keepdims=True)
    var = jnp.mean((t - mu) * (t - mu), axis=0, keepdims=True)
    r = jnp.maximum(
        (t - mu) * lax.rsqrt(var + 1e-5) * g_ref[...] + be_ref[...], 0.0)
    o_ref[...] = (jnp.dot(r, wo_ref[...], preferred_element_type=jnp.float32)
                  + bo_ref[...])


@jax.jit
def kernel(x, edge_index, Wi, bi, Wl0, bl0, Wr0, g0, be0, Wl1, bl1, Wr1, g1,
           be1, Wo, bo):
    N, D = x.shape
    H = Wi.shape[1]
    O = Wo.shape[1]
    E = edge_index.shape[1]
    NW = NC * NS
    assert E % (NW * K) == 0 and (E // (NW * K)) % 4 == 0
    NCHUNK = E // (NW * K)

    er = edge_index.reshape(2, NW, NCHUNK, K).transpose(1, 2, 0, 3)
    ones = jnp.ones((K, 16), jnp.float32)

    agg_deg = _make_sc_agg(N, H, NCHUNK, with_deg=True)
    agg_only = _make_sc_agg(N, H, NCHUNK, with_deg=False)

    proj = pl.pallas_call(
        _proj_relu_body,
        out_shape=jax.ShapeDtypeStruct((N, H), jnp.float32))
    sage_bn = pl.pallas_call(
        _sage_bn_body,
        out_shape=jax.ShapeDtypeStruct((N, H), jnp.float32))
    sage_bn_proj = pl.pallas_call(
        _sage_bn_proj_body,
        out_shape=jax.ShapeDtypeStruct((N, O), jnp.float32))

    h0 = proj(x, Wi, bi.reshape(1, H))
    part0, degp = agg_deg(h0, er, ones)
    h1 = sage_bn(part0, degp, h0, Wl0, Wr0, bl0.reshape(1, H),
                 g0.reshape(1, H), be0.reshape(1, H))
    (part1,) = agg_only(h1, er, ones)
    return sage_bn_proj(part1, degp, h1, Wl1, Wr1, bl1.reshape(1, H),
                        g1.reshape(1, H), be1.reshape(1, H), Wo,
                        bo.reshape(1, O))
